# baseline (speedup 1.0000x reference)
import jax
import jax.numpy as jnp
from jax import lax
from jax.experimental import pallas as pl
from jax.experimental.pallas import tpu as pltpu
from jax.experimental.pallas import tpu_sc as plsc

B = 4
N_NODES = 2048
S = 8192
D = 1024
L = 16

NW = 32
TOKENS = B * S
TPW = TOKENS // NW
CHUNK = 16
NCHUNK = TPW // CHUNK
SLICES_PER_ROW = D // L
NBUF = 2


def _sc_body(node_hbm, tok_hbm, idx_hbm, out_hbm, idx_all,
             in_v0, in_v1, out_v0, out_v1,
             sem_i0, sem_i1, sem_o0, sem_o1):
    in_v = (in_v0, in_v1)
    out_v = (out_v0, out_v1)
    sem_i = (sem_i0, sem_i1)
    sem_o = (sem_o0, sem_o1)

    wid = lax.axis_index("s") * 2 + lax.axis_index("c")
    base = wid * TPW
    row_off = (base // S) * N_NODES

    pltpu.sync_copy(idx_hbm.at[pl.ds(base, TPW)], idx_all)

    def adjust(j, carry):
        sl = pl.ds(j * L, L)
        idx_all[sl] = idx_all[sl] + row_off
        return carry

    lax.fori_loop(0, TPW // L, adjust, None)

    def issue_in(ci, b):
        pltpu.async_copy(
            node_hbm.at[idx_all.at[pl.ds(ci * CHUNK, CHUNK)]],
            in_v[b].at[pl.ds(0, CHUNK)], sem_i[b])
        pltpu.async_copy(
            tok_hbm.at[pl.ds(base + ci * CHUNK, CHUNK)],
            in_v[b].at[pl.ds(CHUNK, CHUNK)], sem_i[b])

    for b in range(NBUF):
        issue_in(b, b)

    def outer(g, carry):
        for b in range(NBUF):
            ci = g * NBUF + b
            tbase = base + ci * CHUNK
            pltpu.make_async_copy(
                node_hbm.at[pl.ds(0, 2 * CHUNK)], in_v[b], sem_i[b]).wait()

            @pl.when(g > 0)
            def _wait_out():
                pltpu.make_async_copy(
                    out_v[b], out_hbm.at[pl.ds(0, CHUNK)], sem_o[b]).wait()

            def row_add(r, c2):
                for j in range(SLICES_PER_ROW):
                    sl = pl.ds(j * L, L)
                    out_v[b][r, sl] = in_v[b][CHUNK + r, sl] + in_v[b][r, sl]
                return c2

            lax.fori_loop(0, CHUNK, row_add, None)

            @pl.when(g < NCHUNK // NBUF - 1)
            def _prefetch():
                issue_in(ci + NBUF, b)

            pltpu.async_copy(out_v[b], out_hbm.at[pl.ds(tbase, CHUNK)], sem_o[b])
        return carry

    lax.fori_loop(0, NCHUNK // NBUF, outer, None)

    for b in range(NBUF):
        pltpu.make_async_copy(
            out_v[b], out_hbm.at[pl.ds(0, CHUNK)], sem_o[b]).wait()


@jax.jit
def _distribute(node_flat, tok_flat, idx_flat):
    mesh = plsc.VectorSubcoreMesh(core_axis_name="c", subcore_axis_name="s")
    f = pl.kernel(
        _sc_body,
        mesh=mesh,
        out_type=jax.ShapeDtypeStruct((TOKENS, D), jnp.float32),
        scratch_types=[
            pltpu.VMEM((TPW,), jnp.int32),
            pltpu.VMEM((2 * CHUNK, D), jnp.float32),
            pltpu.VMEM((2 * CHUNK, D), jnp.float32),
            pltpu.VMEM((CHUNK, D), jnp.float32),
            pltpu.VMEM((CHUNK, D), jnp.float32),
            pltpu.SemaphoreType.DMA,
            pltpu.SemaphoreType.DMA,
            pltpu.SemaphoreType.DMA,
            pltpu.SemaphoreType.DMA,
        ],
    )
    return f(node_flat, tok_flat, idx_flat)


def kernel(node_embeddings, token_embeddings, token_to_node):
    node_flat = node_embeddings.reshape(B * N_NODES, D)
    tok_flat = token_embeddings.reshape(TOKENS, D)
    idx_flat = token_to_node.astype(jnp.int32).reshape(TOKENS)
    out = _distribute(node_flat, tok_flat, idx_flat)
    return out.reshape(B, S, D)


# speedup vs baseline: 1.2127x; 1.2127x over previous
import jax
import jax.numpy as jnp
from jax import lax
from jax.experimental import pallas as pl
from jax.experimental.pallas import tpu as pltpu
from jax.experimental.pallas import tpu_sc as plsc

B = 4
N_NODES = 2048
S = 8192
D = 1024
L = 16

NW = 32
TOKENS = B * S
TPW = TOKENS // NW
CHUNK = 16
NCHUNK = TPW // CHUNK
SLICES_PER_ROW = D // L
NBUF = 2


def _sc_body(node_hbm, tok_hbm, idx_hbm, out_hbm, idx_all,
             node_v0, node_v1, tok_v0, tok_v1, out_v0, out_v1,
             sem_n0, sem_n1, sem_t0, sem_t1, sem_o0, sem_o1):
    node_v = (node_v0, node_v1)
    tok_v = (tok_v0, tok_v1)
    out_v = (out_v0, out_v1)
    sem_n = (sem_n0, sem_n1)
    sem_t = (sem_t0, sem_t1)
    sem_o = (sem_o0, sem_o1)

    wid = lax.axis_index("s") * 2 + lax.axis_index("c")
    base = wid * TPW
    row_off = (base // S) * N_NODES

    pltpu.sync_copy(idx_hbm.at[pl.ds(base, TPW)], idx_all)

    def adjust(j, carry):
        sl = pl.ds(j * L, L)
        idx_all[sl] = idx_all[sl] + row_off
        return carry

    lax.fori_loop(0, TPW // L, adjust, None)

    def issue_in(ci, b):
        pltpu.async_copy(
            node_hbm.at[idx_all.at[pl.ds(ci * CHUNK, CHUNK)]], node_v[b], sem_n[b])
        pltpu.async_copy(
            tok_hbm.at[pl.ds(base + ci * CHUNK, CHUNK)], tok_v[b], sem_t[b])

    for b in range(NBUF):
        issue_in(b, b)

    def outer(g, carry):
        for b in range(NBUF):
            ci = g * NBUF + b
            tbase = base + ci * CHUNK
            pltpu.make_async_copy(
                node_hbm.at[pl.ds(0, CHUNK)], node_v[b], sem_n[b]).wait()
            pltpu.make_async_copy(
                tok_hbm.at[pl.ds(0, CHUNK)], tok_v[b], sem_t[b]).wait()

            @pl.when(g > 0)
            def _wait_out():
                pltpu.make_async_copy(
                    out_v[b], out_hbm.at[pl.ds(0, CHUNK)], sem_o[b]).wait()

            def row_add(r, c2):
                for j in range(SLICES_PER_ROW):
                    sl = pl.ds(j * L, L)
                    out_v[b][r, sl] = tok_v[b][r, sl] + node_v[b][r, sl]
                return c2

            lax.fori_loop(0, CHUNK, row_add, None)

            pltpu.async_copy(out_v[b], out_hbm.at[pl.ds(tbase, CHUNK)], sem_o[b])

            @pl.when(g < NCHUNK // NBUF - 1)
            def _prefetch():
                issue_in(ci + NBUF, b)
        return carry

    lax.fori_loop(0, NCHUNK // NBUF, outer, None)

    for b in range(NBUF):
        pltpu.make_async_copy(
            out_v[b], out_hbm.at[pl.ds(0, CHUNK)], sem_o[b]).wait()


@jax.jit
def _distribute(node_flat, tok_flat, idx_flat):
    mesh = plsc.VectorSubcoreMesh(core_axis_name="c", subcore_axis_name="s")
    f = pl.kernel(
        _sc_body,
        mesh=mesh,
        out_type=jax.ShapeDtypeStruct((TOKENS, D), jnp.float32),
        scratch_types=[
            pltpu.VMEM((TPW,), jnp.int32),
            pltpu.VMEM((CHUNK, D), jnp.float32),
            pltpu.VMEM((CHUNK, D), jnp.float32),
            pltpu.VMEM((CHUNK, D), jnp.float32),
            pltpu.VMEM((CHUNK, D), jnp.float32),
            pltpu.VMEM((CHUNK, D), jnp.float32),
            pltpu.VMEM((CHUNK, D), jnp.float32),
            pltpu.SemaphoreType.DMA,
            pltpu.SemaphoreType.DMA,
            pltpu.SemaphoreType.DMA,
            pltpu.SemaphoreType.DMA,
            pltpu.SemaphoreType.DMA,
            pltpu.SemaphoreType.DMA,
        ],
    )
    return f(node_flat, tok_flat, idx_flat)


def kernel(node_embeddings, token_embeddings, token_to_node):
    node_flat = node_embeddings.reshape(B * N_NODES, D)
    tok_flat = token_embeddings.reshape(TOKENS, D)
    idx_flat = token_to_node.astype(jnp.int32).reshape(TOKENS)
    out = _distribute(node_flat, tok_flat, idx_flat)
    return out.reshape(B, S, D)
